# baseline (speedup 1.0000x reference)
import functools

import jax
import jax.numpy as jnp
from jax import lax
from jax.experimental import pallas as pl
from jax.experimental.pallas import tpu as pltpu
from jax.experimental.pallas import tpu_sc as plsc

B_, S_, D_ = 4, 8192, 1024
NC, NS, L = 2, 16, 16
NW = NC * NS
ROWS_PER_W = S_ // NW
C = 32
CHUNKS = ROWS_PER_W // C
CD = C * D_
T = CHUNKS * B_
UNROLL = 8

_mesh = plsc.VectorSubcoreMesh(core_axis_name="c", subcore_axis_name="s")


@functools.partial(
    pl.kernel,
    mesh=_mesh,
    out_type=jax.ShapeDtypeStruct((B_ * S_ * D_,), jnp.float32),
    scratch_types=[
        pltpu.VMEM((2 * CD,), jnp.float32),
        pltpu.VMEM((CD,), jnp.float32),
        pltpu.SemaphoreType.DMA,
        pltpu.SemaphoreType.DMA,
        pltpu.SemaphoreType.DMA,
        pltpu.SemaphoreType.DMA,
        pltpu.SemaphoreType.DMA,
        pltpu.SemaphoreType.DMA,
    ],
)
def _sc_add(x_hbm, emb_hbm, out_hbm, xbuf, ebuf,
            xs0, xs1, es0, es1, ss0, ss1):
    xsem = (xs0, xs1)
    esem = (es0, es1)
    ssem = (ss0, ss1)
    wid = lax.axis_index("s") * NC + lax.axis_index("c")
    base = wid * (ROWS_PER_W * D_)

    xloads = [None] * T
    eloads = [None] * CHUNKS
    stores = [None] * T

    def item_offsets(t):
        j, b = divmod(t, B_)
        off = b * (S_ * D_) + base + j * CD
        return j, b, off

    for t in range(T + 1):
        if t < T:
            j, b, off = item_offsets(t)
            if t >= 2:
                stores[t - 2].wait()
            xloads[t] = pltpu.async_copy(
                x_hbm.at[pl.ds(off, CD)],
                xbuf.at[pl.ds((t % 2) * CD, CD)],
                xsem[t % 2])
            if t == 0:
                eloads[0] = pltpu.async_copy(
                    emb_hbm.at[pl.ds(base, CD)],
                    ebuf,
                    esem[0])
        if t >= 1:
            tp = t - 1
            j, b, off = item_offsets(tp)
            xloads[tp].wait()
            if b == 0:
                eloads[j].wait()
            xoff = (tp % 2) * CD
            eoff = 0

            @plsc.parallel_loop(0, CD, L, unroll=UNROLL)
            def add_body(i, xoff=xoff, eoff=eoff):
                xs = pl.ds(xoff + i, L)
                es = pl.ds(eoff + i, L)
                xbuf[xs] = xbuf[xs] + ebuf[es]
            stores[tp] = pltpu.async_copy(
                xbuf.at[pl.ds(xoff, CD)],
                out_hbm.at[pl.ds(off, CD)],
                ssem[tp % 2])
            if b == B_ - 1 and j + 1 < CHUNKS:
                eloads[j + 1] = pltpu.async_copy(
                    emb_hbm.at[pl.ds(base + (j + 1) * CD, CD)],
                    ebuf,
                    esem[(j + 1) % 2])

    stores[T - 2].wait()
    stores[T - 1].wait()


def kernel(x, emb_table):
    b, s, d = x.shape
    out = _sc_add(x.reshape(-1), emb_table[:s].reshape(-1))
    return out.reshape(b, s, d)


# speedup vs baseline: 4.4186x; 4.4186x over previous
import jax
import jax.numpy as jnp
from jax.experimental import pallas as pl
from jax.experimental.pallas import tpu as pltpu

_BS = 512


def _add_kernel(x_ref, emb_ref, out_ref):
    out_ref[...] = x_ref[...] + emb_ref[...][None, :, :]


def kernel(x, emb_table):
    B, S, D = x.shape
    grid = (S // _BS,)
    return pl.pallas_call(
        _add_kernel,
        grid=grid,
        in_specs=[
            pl.BlockSpec((B, _BS, D), lambda i: (0, i, 0)),
            pl.BlockSpec((_BS, D), lambda i: (i, 0)),
        ],
        out_specs=pl.BlockSpec((B, _BS, D), lambda i: (0, i, 0)),
        out_shape=jax.ShapeDtypeStruct((B, S, D), x.dtype),
        compiler_params=pltpu.CompilerParams(
            dimension_semantics=("parallel",),
        ),
    )(x, emb_table[:S])


# speedup vs baseline: 4.4620x; 1.0098x over previous
import jax
import jax.numpy as jnp
from jax.experimental import pallas as pl
from jax.experimental.pallas import tpu as pltpu

_BS = 2048


def _add_kernel(x_ref, emb_ref, out_ref):
    out_ref[...] = x_ref[...] + emb_ref[...][None, :, :]


def kernel(x, emb_table):
    B, S, D = x.shape
    grid = (S // _BS, B)
    return pl.pallas_call(
        _add_kernel,
        grid=grid,
        in_specs=[
            pl.BlockSpec((1, _BS, D), lambda i, b: (b, i, 0)),
            pl.BlockSpec((_BS, D), lambda i, b: (i, 0)),
        ],
        out_specs=pl.BlockSpec((1, _BS, D), lambda i, b: (b, i, 0)),
        out_shape=jax.ShapeDtypeStruct((B, S, D), x.dtype),
        compiler_params=pltpu.CompilerParams(
            dimension_semantics=("arbitrary", "arbitrary"),
        ),
    )(x, emb_table[:S])
